# baseline (speedup 1.0000x reference)
import jax
import jax.numpy as jnp
from jax import lax
from jax.experimental import pallas as pl
from jax.experimental.pallas import tpu as pltpu
from jax.experimental.pallas import tpu_sc as plsc

DIM = 300
DPAD = 384
B = 4096 * 200
NC, NS = 2, 16
NW = NC * NS
BPW = B // NW
CHUNK = 128
NCHUNK = BPW // CHUNK


def _emb_body(table_hbm, idx_hbm, out_hbm, idx_v, rows_v, sem):
    wid = lax.axis_index("s") * NC + lax.axis_index("c")
    base = wid * BPW

    def body(g, carry):
        off = base + g * CHUNK
        pltpu.sync_copy(idx_hbm.at[pl.ds(off, CHUNK)], idx_v)
        pltpu.async_copy(table_hbm.at[idx_v], rows_v, sem).wait()
        pltpu.sync_copy(rows_v, out_hbm.at[pl.ds(off, CHUNK)])
        return carry

    lax.fori_loop(0, NCHUNK, body, 0)


def kernel(table, idxes):
    idx_flat = idxes.reshape(-1).astype(jnp.int32)
    table_pad = jnp.pad(table, ((0, 0), (0, DPAD - DIM)))
    mesh = plsc.VectorSubcoreMesh(core_axis_name="c", subcore_axis_name="s")
    out_pad = pl.kernel(
        _emb_body,
        out_type=jax.ShapeDtypeStruct((B, DPAD), jnp.float32),
        mesh=mesh,
        scratch_types=[
            pltpu.VMEM((CHUNK,), jnp.int32),
            pltpu.VMEM((CHUNK, DPAD), jnp.float32),
            pltpu.SemaphoreType.DMA,
        ],
    )(table_pad, idx_flat)
    return out_pad[:, :DIM].reshape(idxes.shape + (DIM,))


# speedup vs baseline: 1.8115x; 1.8115x over previous
import jax
import jax.numpy as jnp
from jax import lax
from jax.experimental import pallas as pl
from jax.experimental.pallas import tpu as pltpu
from jax.experimental.pallas import tpu_sc as plsc

DIM = 300
DPAD = 384
B = 4096 * 200
NC, NS = 2, 16
NW = NC * NS
BPW = B // NW
CHUNK = 128
NCHUNK = BPW // CHUNK


def _emb_body(table_hbm, idx_hbm, out_hbm, idx_v, rows_v, sem):
    wid = lax.axis_index("s") * NC + lax.axis_index("c")
    base = wid * BPW

    def body(g, carry):
        off = base + g * CHUNK
        pltpu.sync_copy(idx_hbm.at[pl.ds(off, CHUNK)], idx_v)
        pltpu.async_copy(table_hbm.at[idx_v], rows_v, sem).wait()
        pltpu.sync_copy(rows_v, out_hbm.at[pl.ds(off, CHUNK)])
        return carry

    lax.fori_loop(0, NCHUNK, body, 0)


_PAD_ROWS = 1224


def _pad_body(x_ref, o_ref):
    o_ref[...] = jnp.concatenate(
        [x_ref[...], jnp.zeros((_PAD_ROWS, DPAD - DIM), jnp.float32)], axis=1)


def _pad_table(table):
    v = table.shape[0]
    return pl.pallas_call(
        _pad_body,
        grid=(v // _PAD_ROWS,),
        in_specs=[pl.BlockSpec((_PAD_ROWS, DIM), lambda i: (i, 0))],
        out_specs=pl.BlockSpec((_PAD_ROWS, DPAD), lambda i: (i, 0)),
        out_shape=jax.ShapeDtypeStruct((v, DPAD), jnp.float32),
    )(table)


def kernel(table, idxes):
    idx_flat = idxes.reshape(-1).astype(jnp.int32)
    table_pad = _pad_table(table)
    mesh = plsc.VectorSubcoreMesh(core_axis_name="c", subcore_axis_name="s")
    out_pad = pl.kernel(
        _emb_body,
        out_type=jax.ShapeDtypeStruct((B, DPAD), jnp.float32),
        mesh=mesh,
        scratch_types=[
            pltpu.VMEM((CHUNK,), jnp.int32),
            pltpu.VMEM((CHUNK, DPAD), jnp.float32),
            pltpu.SemaphoreType.DMA,
        ],
    )(table_pad, idx_flat)
    return out_pad[:, :DIM].reshape(idxes.shape + (DIM,))


# speedup vs baseline: 1.8915x; 1.0442x over previous
import jax
import jax.numpy as jnp
from jax import lax
from jax.experimental import pallas as pl
from jax.experimental.pallas import tpu as pltpu
from jax.experimental.pallas import tpu_sc as plsc

DIM = 300
D1 = 256
D2 = DIM - D1
TPAD = 128
B = 4096 * 200
NC, NS = 2, 16
NW = NC * NS
BPW = B // NW
CHUNK = 128
NCHUNK = BPW // CHUNK

_TAIL_ROWS = 1224


def _tail_body(x_ref, o_ref):
    lane = lax.broadcasted_iota(jnp.int32, (_TAIL_ROWS, TPAD), 1)
    o_ref[...] = jnp.where(lane < D2, x_ref[...], 0.0)


def _make_tail(table):
    v = table.shape[0]
    return pl.pallas_call(
        _tail_body,
        grid=(v // _TAIL_ROWS,),
        in_specs=[pl.BlockSpec((_TAIL_ROWS, TPAD), lambda i: (i, 2))],
        out_specs=pl.BlockSpec((_TAIL_ROWS, TPAD), lambda i: (i, 0)),
        out_shape=jax.ShapeDtypeStruct((v, TPAD), jnp.float32),
    )(table)


def _emb_body(table_hbm, tail_hbm, idx_hbm, out_hbm,
              idx_v, rows_v, tail_v, sem, sem2):
    wid = lax.axis_index("s") * NC + lax.axis_index("c")
    base = wid * BPW
    iota = lax.iota(jnp.int32, 16)
    tail_mask = iota < (DIM - D1 - 32)

    def body(g, carry):
        off = base + g * CHUNK
        pltpu.sync_copy(idx_hbm.at[pl.ds(off, CHUNK)], idx_v)
        cp1 = pltpu.async_copy(table_hbm.at[idx_v, pl.ds(0, D1)],
                               rows_v.at[:, pl.ds(0, D1)], sem)
        cp2 = pltpu.async_copy(tail_hbm.at[idx_v], tail_v, sem2)
        cp1.wait()
        cp2.wait()

        def mrow(b, c):
            rows_v[b, pl.ds(D1, 16)] = tail_v[b, pl.ds(0, 16)]
            rows_v[b, pl.ds(D1 + 16, 16)] = tail_v[b, pl.ds(16, 16)]
            x2 = tail_v[b, pl.ds(32, 16)]
            plsc.store_scatter(rows_v, [jnp.full((16,), b, jnp.int32),
                                        D1 + 32 + iota], x2, mask=tail_mask)
            return c

        lax.fori_loop(0, CHUNK, mrow, 0)
        pltpu.sync_copy(rows_v, out_hbm.at[pl.ds(off, CHUNK)])
        return carry

    lax.fori_loop(0, NCHUNK, body, 0)


def kernel(table, idxes):
    idx_flat = idxes.reshape(-1).astype(jnp.int32)
    tail = _make_tail(table)
    mesh = plsc.VectorSubcoreMesh(core_axis_name="c", subcore_axis_name="s")
    out = pl.kernel(
        _emb_body,
        out_type=jax.ShapeDtypeStruct((B, DIM), jnp.float32),
        mesh=mesh,
        compiler_params=pltpu.CompilerParams(needs_layout_passes=False),
        scratch_types=[
            pltpu.VMEM((CHUNK,), jnp.int32),
            pltpu.VMEM((CHUNK, DIM), jnp.float32),
            pltpu.VMEM((CHUNK, TPAD), jnp.float32),
            pltpu.SemaphoreType.DMA,
            pltpu.SemaphoreType.DMA,
        ],
    )(table, tail, idx_flat)
    return out.reshape(idxes.shape + (DIM,))
